# baseline (speedup 1.0000x reference)
import functools

import jax
import jax.numpy as jnp
from jax import lax
from jax.experimental import pallas as pl
from jax.experimental.pallas import tpu as pltpu
from jax.experimental.pallas import tpu_sc as plsc

_B, _N, _D = 4, 8192, 1024
_NC, _NS = 2, 16
_NW = _NC * _NS
_RPW = _N // _NW
_C = 8
_CHUNKS = _RPW // _C
_DV = _D // 16
_L = 16


def _sc_body(x_hbm, pe_hbm, out_hbm, *refs):
    xbs = [[refs[p * _B + b] for b in range(_B)] for p in range(2)]
    pebs = list(refs[8:10])
    in_sems = [[refs[10 + p * _B + b] for b in range(_B)] for p in range(2)]
    out_sems = [[refs[18 + p * _B + b] for b in range(_B)] for p in range(2)]
    pe_sems = list(refs[26:28])

    wid = lax.axis_index("s") * _NC + lax.axis_index("c")
    n_base = wid * _RPW

    def rows(j):
        return pl.ds(n_base + j * _C, _C)

    def in_copy(j, b, p):
        return pltpu.make_async_copy(
            x_hbm.at[b, rows(j), :], xbs[p][b], in_sems[p][b])

    def out_copy(j, b, p):
        return pltpu.make_async_copy(
            xbs[p][b], out_hbm.at[b, rows(j), :], out_sems[p][b])

    def pe_copy(j, p):
        return pltpu.make_async_copy(pe_hbm.at[rows(j), :], pebs[p], pe_sems[p])

    def add_chunk(p):
        peb = pebs[p]

        for b in range(_B):
            xb = xbs[p][b]

            @plsc.parallel_loop(0, _C * _DV, unroll=16)
            def _vec(j):
                r = j // _DV
                off = (j % _DV) * _L
                plsc.addupdate(xb.at[r, pl.ds(off, _L)],
                               peb[r, pl.ds(off, _L)])

    pe_copy(0, 0).start()
    pe_copy(1, 1).start()
    for b in range(_B):
        in_copy(0, b, 0).start()
    for b in range(_B):
        in_copy(1, b, 1).start()

    @pl.loop(0, _CHUNKS // 2)
    def _pair(u):
        j0 = 2 * u
        j1 = 2 * u + 1
        last = u == _CHUNKS // 2 - 1

        @pl.when(u > 0)
        def _():
            for b in range(_B):
                out_copy(j0 - 1, b, 1).wait()
                in_copy(j1, b, 1).start()
        pe_copy(j0, 0).wait()
        for b in range(_B):
            in_copy(j0, b, 0).wait()
        add_chunk(0)
        for b in range(_B):
            out_copy(j0, b, 0).start()

        @pl.when(jnp.logical_not(last))
        def _():
            pe_copy(j0 + 2, 0).start()

        for b in range(_B):
            out_copy(j0, b, 0).wait()
            @pl.when(jnp.logical_not(last))
            def _():
                in_copy(j0 + 2, b, 0).start()
        pe_copy(j1, 1).wait()
        for b in range(_B):
            in_copy(j1, b, 1).wait()
        add_chunk(1)
        for b in range(_B):
            out_copy(j1, b, 1).start()

        @pl.when(jnp.logical_not(last))
        def _():
            pe_copy(j1 + 2, 1).start()

    for b in range(_B):
        out_copy(_CHUNKS - 1, b, 1).wait()


_sc_add = functools.partial(
    pl.kernel,
    out_type=jax.ShapeDtypeStruct((_B, _N, _D), jnp.float32),
    mesh=plsc.VectorSubcoreMesh(
        core_axis_name="c", subcore_axis_name="s",
        num_cores=_NC, num_subcores=_NS,
    ),
    scratch_types=(
        [pltpu.VMEM((_C, _D), jnp.float32) for _ in range(10)]
        + [pltpu.SemaphoreType.DMA for _ in range(18)]
    ),
)(_sc_body)


def kernel(x, pos_emb):
    B, N, D = x.shape
    return _sc_add(x, pos_emb[:N])


# speedup vs baseline: 1.0125x; 1.0125x over previous
import functools

import jax
import jax.numpy as jnp
from jax import lax
from jax.experimental import pallas as pl
from jax.experimental.pallas import tpu as pltpu
from jax.experimental.pallas import tpu_sc as plsc

_B, _N, _D = 4, 8192, 1024
_NC, _NS = 2, 16
_NW = _NC * _NS
_RPW = _N // _NW
_C = 8
_CHUNKS = _RPW // _C
_DV = _D // 16
_L = 16


def _sc_body(x_hbm, pe_hbm, out_hbm, *refs):
    xbs = [[refs[p * _B + b] for b in range(_B)] for p in range(2)]
    pebs = list(refs[8:10])
    in_sems = [[refs[10 + p * _B + b] for b in range(_B)] for p in range(2)]
    out_sems = [[refs[18 + p * _B + b] for b in range(_B)] for p in range(2)]
    pe_sems = list(refs[26:28])

    wid = lax.axis_index("s") * _NC + lax.axis_index("c")
    n_base = wid * _RPW

    def rows(j):
        return pl.ds(n_base + j * _C, _C)

    def in_copy(j, b, p):
        return pltpu.make_async_copy(
            x_hbm.at[b, rows(j), :], xbs[p][b], in_sems[p][b])

    def out_copy(j, b, p):
        return pltpu.make_async_copy(
            xbs[p][b], out_hbm.at[b, rows(j), :], out_sems[p][b])

    def pe_copy(j, p):
        return pltpu.make_async_copy(pe_hbm.at[rows(j), :], pebs[p], pe_sems[p])

    def add_chunk(j, p):
        peb = pebs[p]

        for b in range(_B):
            xb = xbs[p][b]
            in_copy(j, b, p).wait()

            @plsc.parallel_loop(0, _C * _DV, unroll=8)
            def _vec(v):
                r = v // _DV
                off = (v % _DV) * _L
                plsc.addupdate(xb.at[r, pl.ds(off, _L)],
                               peb[r, pl.ds(off, _L)])

    pe_copy(0, 0).start()
    pe_copy(1, 1).start()
    for b in range(_B):
        in_copy(0, b, 0).start()
    for b in range(_B):
        in_copy(1, b, 1).start()

    @pl.loop(0, _CHUNKS // 2)
    def _pair(u):
        j0 = 2 * u
        j1 = 2 * u + 1
        last = u == _CHUNKS // 2 - 1

        @pl.when(u > 0)
        def _():
            for b in range(_B):
                out_copy(j0 - 1, b, 1).wait()
                in_copy(j1, b, 1).start()
        pe_copy(j0, 0).wait()
        add_chunk(j0, 0)
        for b in range(_B):
            out_copy(j0, b, 0).start()

        @pl.when(jnp.logical_not(last))
        def _():
            pe_copy(j0 + 2, 0).start()

        for b in range(_B):
            out_copy(j0, b, 0).wait()
            @pl.when(jnp.logical_not(last))
            def _():
                in_copy(j0 + 2, b, 0).start()
        pe_copy(j1, 1).wait()
        add_chunk(j1, 1)
        for b in range(_B):
            out_copy(j1, b, 1).start()

        @pl.when(jnp.logical_not(last))
        def _():
            pe_copy(j1 + 2, 1).start()

    for b in range(_B):
        out_copy(_CHUNKS - 1, b, 1).wait()


_sc_add = functools.partial(
    pl.kernel,
    out_type=jax.ShapeDtypeStruct((_B, _N, _D), jnp.float32),
    mesh=plsc.VectorSubcoreMesh(
        core_axis_name="c", subcore_axis_name="s",
        num_cores=_NC, num_subcores=_NS,
    ),
    scratch_types=(
        [pltpu.VMEM((_C, _D), jnp.float32) for _ in range(10)]
        + [pltpu.SemaphoreType.DMA for _ in range(18)]
    ),
)(_sc_body)


def kernel(x, pos_emb):
    B, N, D = x.shape
    return _sc_add(x, pos_emb[:N])
